# baseline (speedup 1.0000x reference)
import functools

import jax
import jax.numpy as jnp
from jax import lax
from jax.experimental import pallas as pl
from jax.experimental.pallas import tpu as pltpu
from jax.experimental.pallas import tpu_sc as plsc

_L = 16


def _tree_sum(terms):
    while len(terms) > 1:
        nxt = [terms[i] + terms[i + 1] for i in range(0, len(terms) - 1, 2)]
        if len(terms) % 2:
            nxt.append(terms[-1])
        terms = nxt
    return terms[0]


def _tile_body(nc, rpw, chunk, d, m,
               user_hbm, item_hbm, meta_hbm, utab_hbm, itab_hbm,
               w_hbm, mw_hbm, bias_hbm, out_hbm,
               idx_u, idx_i, u0, i0, m0, u1, i1, m1,
               w_v, mw_v, bias_v, out_v, sem0, sem1):
    wid = lax.axis_index("s") * nc + lax.axis_index("c")
    base = pl.multiple_of(wid * rpw, rpw)

    staging = (
        pltpu.async_copy(user_hbm.at[pl.ds(base, rpw)], idx_u, sem0),
        pltpu.async_copy(item_hbm.at[pl.ds(base, rpw)], idx_i, sem0),
        pltpu.async_copy(w_hbm, w_v, sem0),
        pltpu.async_copy(mw_hbm, mw_v, sem0),
        pltpu.async_copy(bias_hbm, bias_v, sem0),
    )
    for cp in staging:
        cp.wait()

    wv = [w_v[pl.ds(k * _L, _L)] for k in range(d // _L)]
    mwv = [mw_v[pl.ds(k * _L, _L)] for k in range(m // _L)]
    bias = bias_v[...]

    def fill(i, carry):
        out_v[pl.ds(i * _L, _L)] = bias
        return carry
    lax.fori_loop(0, rpw // _L, fill, 0)

    nchunk = rpw // chunk
    ngroup = chunk // _L
    bufs = [(u0, i0, m0), (u1, i1, m1)]
    sems = [sem0, sem1]

    def start(c):
        cb = c * chunk
        ub, ib, mb = bufs[c % 2]
        sem = sems[c % 2]
        return (
            pltpu.async_copy(utab_hbm.at[idx_u.at[pl.ds(cb, chunk)]], ub, sem),
            pltpu.async_copy(itab_hbm.at[idx_i.at[pl.ds(cb, chunk)]], ib, sem),
            pltpu.async_copy(meta_hbm.at[pl.ds(base + cb, chunk), :], mb, sem),
        )

    copies = {0: start(0)}
    for c in range(nchunk):
        if c + 1 < nchunk:
            copies[c + 1] = start(c + 1)
        for cp in copies.pop(c):
            cp.wait()
        ub, ib, mb = bufs[c % 2]
        cb = c * chunk

        def group(g, carry, ub=ub, ib=ib, mb=mb, cb=cb):
            gb = g * _L
            half = _L // 2
            for h in range(2):
                hb = gb + h * half
                accs = [None] * half
                for k in range(d // _L):
                    for l in range(half):
                        t = ub[hb + l, pl.ds(k * _L, _L)] * ib[hb + l, pl.ds(k * _L, _L)] * wv[k]
                        accs[l] = t if k == 0 else accs[l] + t
                for k in range(m // _L):
                    for l in range(half):
                        accs[l] = accs[l] + mb[hb + l, pl.ds(k * _L, _L)] * mwv[k]
                for l in range(half):
                    row = cb + hb + l
                    plsc.addupdate_scatter(out_v, [jnp.full((_L,), row, jnp.int32)], accs[l])
            return carry

        lax.fori_loop(0, ngroup, group, 0)

    pltpu.sync_copy(out_v, out_hbm.at[pl.ds(base, rpw)])


def kernel(user, item, item_metadata, user_table, item_table,
           comb_w, comb_b, meta_w, meta_b, global_bias):
    b = user.shape[0]
    d = user_table.shape[1]
    m = item_metadata.shape[1]
    info = plsc.get_sparse_core_info()
    nc, ns = info.num_cores, info.num_subcores
    nw = nc * ns
    rpw = b // nw
    chunk = 128

    w = comb_w.reshape(d)
    mw = meta_w.reshape(m)
    bias = jnp.broadcast_to(comb_b + meta_b + global_bias, (_L,)).astype(jnp.float32)

    mesh = plsc.VectorSubcoreMesh(core_axis_name="c", subcore_axis_name="s")
    kfn = pl.kernel(
        functools.partial(_tile_body, nc, rpw, chunk, d, m),
        mesh=mesh,
        compiler_params=pltpu.CompilerParams(needs_layout_passes=False),
        out_type=jax.ShapeDtypeStruct((b,), jnp.float32),
        scratch_types=[
            pltpu.VMEM((rpw,), jnp.int32),
            pltpu.VMEM((rpw,), jnp.int32),
            pltpu.VMEM((chunk, d), jnp.float32),
            pltpu.VMEM((chunk, d), jnp.float32),
            pltpu.VMEM((chunk, m), jnp.float32),
            pltpu.VMEM((chunk, d), jnp.float32),
            pltpu.VMEM((chunk, d), jnp.float32),
            pltpu.VMEM((chunk, m), jnp.float32),
            pltpu.VMEM((d,), jnp.float32),
            pltpu.VMEM((m,), jnp.float32),
            pltpu.VMEM((_L,), jnp.float32),
            pltpu.VMEM((rpw,), jnp.float32),
            pltpu.SemaphoreType.DMA,
            pltpu.SemaphoreType.DMA,
        ],
    )
    return kfn(user, item, item_metadata, user_table, item_table, w, mw, bias)


# speedup vs baseline: 1.1139x; 1.1139x over previous
import functools

import jax
import jax.numpy as jnp
from jax import lax
from jax.experimental import pallas as pl
from jax.experimental.pallas import tpu as pltpu
from jax.experimental.pallas import tpu_sc as plsc

_L = 16


def _tree_sum(terms):
    while len(terms) > 1:
        nxt = [terms[i] + terms[i + 1] for i in range(0, len(terms) - 1, 2)]
        if len(terms) % 2:
            nxt.append(terms[-1])
        terms = nxt
    return terms[0]


def _tile_body(nc, rpw, chunk, d, m,
               user_hbm, item_hbm, meta_hbm, utab_hbm, itab_hbm,
               w_hbm, mw_hbm, bias_hbm, out_hbm,
               idx_u, idx_i, u0, i0, m0, u1, i1, m1,
               w_v, mw_v, bias_v, trans, out_v, sem0, sem1):
    wid = lax.axis_index("s") * nc + lax.axis_index("c")
    base = pl.multiple_of(wid * rpw, rpw)

    staging = (
        pltpu.async_copy(user_hbm.at[pl.ds(base, rpw)], idx_u, sem0),
        pltpu.async_copy(item_hbm.at[pl.ds(base, rpw)], idx_i, sem0),
        pltpu.async_copy(w_hbm, w_v, sem0),
        pltpu.async_copy(mw_hbm, mw_v, sem0),
        pltpu.async_copy(bias_hbm, bias_v, sem0),
    )
    for cp in staging:
        cp.wait()

    wv = [w_v[pl.ds(k * _L, _L)] for k in range(d // _L)]
    mwv = [mw_v[pl.ds(k * _L, _L)] for k in range(m // _L)]
    bias = bias_v[...]
    lane16 = lax.iota(jnp.int32, _L) * _L

    nchunk = rpw // chunk
    ngroup = chunk // _L
    bufs = [(u0, i0, m0), (u1, i1, m1)]
    sems = [sem0, sem1]

    def start(c):
        cb = c * chunk
        ub, ib, mb = bufs[c % 2]
        sem = sems[c % 2]
        return (
            pltpu.async_copy(utab_hbm.at[idx_u.at[pl.ds(cb, chunk)]], ub, sem),
            pltpu.async_copy(itab_hbm.at[idx_i.at[pl.ds(cb, chunk)]], ib, sem),
            pltpu.async_copy(meta_hbm.at[pl.ds(base + cb, chunk), :], mb, sem),
        )

    copies = {0: start(0)}
    for c in range(nchunk):
        if c + 1 < nchunk:
            copies[c + 1] = start(c + 1)
        for cp in copies.pop(c):
            cp.wait()
        ub, ib, mb = bufs[c % 2]
        cb = c * chunk

        def group(g, carry, ub=ub, ib=ib, mb=mb, cb=cb):
            gb = g * _L
            half = _L // 2
            for h in range(2):
                hb = gb + h * half
                accs = [None] * half
                for k in range(d // _L):
                    for l in range(half):
                        t = ub[hb + l, pl.ds(k * _L, _L)] * ib[hb + l, pl.ds(k * _L, _L)] * wv[k]
                        accs[l] = t if k == 0 else accs[l] + t
                for k in range(m // _L):
                    for l in range(half):
                        accs[l] = accs[l] + mb[hb + l, pl.ds(k * _L, _L)] * mwv[k]
                for l in range(half):
                    trans[pl.ds((h * half + l) * _L, _L)] = accs[l]
            cols = [plsc.load_gather(trans, [lane16 + j]) for j in range(_L)]
            out_v[pl.ds(cb + gb, _L)] = _tree_sum(cols) + bias
            return carry

        lax.fori_loop(0, ngroup, group, 0)

    pltpu.sync_copy(out_v, out_hbm.at[pl.ds(base, rpw)])


def kernel(user, item, item_metadata, user_table, item_table,
           comb_w, comb_b, meta_w, meta_b, global_bias):
    b = user.shape[0]
    d = user_table.shape[1]
    m = item_metadata.shape[1]
    info = plsc.get_sparse_core_info()
    nc, ns = info.num_cores, info.num_subcores
    nw = nc * ns
    rpw = b // nw
    chunk = 128

    w = comb_w.reshape(d)
    mw = meta_w.reshape(m)
    bias = jnp.broadcast_to(comb_b + meta_b + global_bias, (_L,)).astype(jnp.float32)

    mesh = plsc.VectorSubcoreMesh(core_axis_name="c", subcore_axis_name="s")
    kfn = pl.kernel(
        functools.partial(_tile_body, nc, rpw, chunk, d, m),
        mesh=mesh,
        compiler_params=pltpu.CompilerParams(needs_layout_passes=False),
        out_type=jax.ShapeDtypeStruct((b,), jnp.float32),
        scratch_types=[
            pltpu.VMEM((rpw,), jnp.int32),
            pltpu.VMEM((rpw,), jnp.int32),
            pltpu.VMEM((chunk, d), jnp.float32),
            pltpu.VMEM((chunk, d), jnp.float32),
            pltpu.VMEM((chunk, m), jnp.float32),
            pltpu.VMEM((chunk, d), jnp.float32),
            pltpu.VMEM((chunk, d), jnp.float32),
            pltpu.VMEM((chunk, m), jnp.float32),
            pltpu.VMEM((d,), jnp.float32),
            pltpu.VMEM((m,), jnp.float32),
            pltpu.VMEM((_L,), jnp.float32),
            pltpu.VMEM((_L * _L,), jnp.float32),
            pltpu.VMEM((rpw,), jnp.float32),
            pltpu.SemaphoreType.DMA,
            pltpu.SemaphoreType.DMA,
        ],
    )
    return kfn(user, item, item_metadata, user_table, item_table, w, mw, bias)


# speedup vs baseline: 1.1740x; 1.0540x over previous
import functools

import jax
import jax.numpy as jnp
from jax import lax
from jax.experimental import pallas as pl
from jax.experimental.pallas import tpu as pltpu
from jax.experimental.pallas import tpu_sc as plsc

_L = 16


def _tree_sum(terms):
    while len(terms) > 1:
        nxt = [terms[i] + terms[i + 1] for i in range(0, len(terms) - 1, 2)]
        if len(terms) % 2:
            nxt.append(terms[-1])
        terms = nxt
    return terms[0]


def _tile_body(nc, rpw, chunk, d, m,
               user_hbm, item_hbm, meta_hbm, utab_hbm, itab_hbm,
               w_hbm, mw_hbm, bias_hbm, out_hbm,
               idx_u, idx_i, u0, i0, m0, u1, i1, m1,
               w_v, mw_v, bias_v, trans, out_v, stg_sem, w_sem, sem0, sem1):
    wid = lax.axis_index("s") * nc + lax.axis_index("c")
    base = pl.multiple_of(wid * rpw, rpw)

    idx_cps = (
        pltpu.async_copy(user_hbm.at[pl.ds(base, rpw)], idx_u, stg_sem),
        pltpu.async_copy(item_hbm.at[pl.ds(base, rpw)], idx_i, stg_sem),
    )
    w_cps = (
        pltpu.async_copy(w_hbm, w_v, w_sem),
        pltpu.async_copy(mw_hbm, mw_v, w_sem),
        pltpu.async_copy(bias_hbm, bias_v, w_sem),
    )
    for cp in idx_cps:
        cp.wait()

    wv = [w_v[pl.ds(k * _L, _L)] for k in range(d // _L)]
    mwv = [mw_v[pl.ds(k * _L, _L)] for k in range(m // _L)]
    bias = bias_v[...]
    lane16 = lax.iota(jnp.int32, _L) * _L

    nchunk = rpw // chunk
    ngroup = chunk // _L
    bufs = [(u0, i0, m0), (u1, i1, m1)]
    sems = [sem0, sem1]

    def start(c, parity):
        cb = pl.multiple_of(c * chunk, chunk)
        ub, ib, mb = bufs[parity]
        sem = sems[parity]
        pltpu.async_copy(utab_hbm.at[idx_u.at[pl.ds(cb, chunk)]], ub, sem)
        pltpu.async_copy(itab_hbm.at[idx_i.at[pl.ds(cb, chunk)]], ib, sem)
        pltpu.async_copy(meta_hbm.at[pl.ds(base + cb, chunk), :], mb, sem)

    def drain(parity):
        ub, ib, mb = bufs[parity]
        sem = sems[parity]
        pltpu.make_async_copy(utab_hbm.at[pl.ds(0, chunk), :], ub, sem).wait()
        pltpu.make_async_copy(itab_hbm.at[pl.ds(0, chunk), :], ib, sem).wait()
        pltpu.make_async_copy(meta_hbm.at[pl.ds(0, chunk), :], mb, sem).wait()

    start(0, 0)
    start(1, 1)
    for cp in w_cps:
        cp.wait()

    def compute_chunk(c, parity):
        ub, ib, mb = bufs[parity]
        cb = pl.multiple_of(c * chunk, chunk)

        def group(g, carry, ub=ub, ib=ib, mb=mb, cb=cb):
            gb = g * _L
            half = _L // 2
            for h in range(2):
                hb = gb + h * half
                accs = [None] * half
                for k in range(d // _L):
                    for l in range(half):
                        t = ub[hb + l, pl.ds(k * _L, _L)] * ib[hb + l, pl.ds(k * _L, _L)] * wv[k]
                        accs[l] = t if k == 0 else accs[l] + t
                for k in range(m // _L):
                    for l in range(half):
                        accs[l] = accs[l] + mb[hb + l, pl.ds(k * _L, _L)] * mwv[k]
                for l in range(half):
                    trans[pl.ds((h * half + l) * _L, _L)] = accs[l]
            cols = [plsc.load_gather(trans, [lane16 + j]) for j in range(_L)]
            out_v[pl.ds(cb + gb, _L)] = _tree_sum(cols) + bias
            return carry

        lax.fori_loop(0, ngroup, group, 0)

    def pair_body(p, carry):
        for parity in range(2):
            c = p * 2 + parity
            drain(parity)

            @pl.when(c + 2 < nchunk)
            def _(c=c, parity=parity):
                start(c + 2, parity)

            compute_chunk(c, parity)
        return carry

    lax.fori_loop(0, nchunk // 2, pair_body, 0)

    pltpu.sync_copy(out_v, out_hbm.at[pl.ds(base, rpw)])


def kernel(user, item, item_metadata, user_table, item_table,
           comb_w, comb_b, meta_w, meta_b, global_bias):
    b = user.shape[0]
    d = user_table.shape[1]
    m = item_metadata.shape[1]
    info = plsc.get_sparse_core_info()
    nc, ns = info.num_cores, info.num_subcores
    nw = nc * ns
    rpw = b // nw
    chunk = 128

    w = comb_w.reshape(d)
    mw = meta_w.reshape(m)
    bias = jnp.broadcast_to(comb_b + meta_b + global_bias, (_L,)).astype(jnp.float32)

    mesh = plsc.VectorSubcoreMesh(core_axis_name="c", subcore_axis_name="s")
    kfn = pl.kernel(
        functools.partial(_tile_body, nc, rpw, chunk, d, m),
        mesh=mesh,
        compiler_params=pltpu.CompilerParams(needs_layout_passes=False),
        out_type=jax.ShapeDtypeStruct((b,), jnp.float32),
        scratch_types=[
            pltpu.VMEM((rpw,), jnp.int32),
            pltpu.VMEM((rpw,), jnp.int32),
            pltpu.VMEM((chunk, d), jnp.float32),
            pltpu.VMEM((chunk, d), jnp.float32),
            pltpu.VMEM((chunk, m), jnp.float32),
            pltpu.VMEM((chunk, d), jnp.float32),
            pltpu.VMEM((chunk, d), jnp.float32),
            pltpu.VMEM((chunk, m), jnp.float32),
            pltpu.VMEM((d,), jnp.float32),
            pltpu.VMEM((m,), jnp.float32),
            pltpu.VMEM((_L,), jnp.float32),
            pltpu.VMEM((_L * _L,), jnp.float32),
            pltpu.VMEM((rpw,), jnp.float32),
            pltpu.SemaphoreType.DMA,
            pltpu.SemaphoreType.DMA,
            pltpu.SemaphoreType.DMA,
            pltpu.SemaphoreType.DMA,
        ],
    )
    return kfn(user, item, item_metadata, user_table, item_table, w, mw, bias)
